# baseline (speedup 1.0000x reference)
import functools

import jax
import jax.numpy as jnp
from jax import lax
from jax.experimental import pallas as pl
from jax.experimental.pallas import tpu as pltpu
from jax.experimental.pallas import tpu_sc as plsc

B = 4096 * 26
NW = 32
BPW = B // NW
K = 128
NCH = BPW // K
NG = K // 16


def _tables_body(a0, a1, a3t, a2pt, t1, t2):
    m1 = jnp.dot(a0[...], a1[...], preferred_element_type=jnp.float32)
    m2q = jnp.dot(a3t[...], a2pt[...], preferred_element_type=jnp.float32)
    t1[...] = (
        m1.reshape(32, 2, 32, 4, 16).transpose(0, 2, 1, 3, 4).reshape(1024, 128)
    )
    t2[...] = (
        m2q.reshape(32, 2, 32, 4, 16).transpose(2, 0, 3, 1, 4).reshape(1024, 128)
    )


def _make_tables(A0, A1, A2, A3):
    return pl.pallas_call(
        _tables_body,
        out_shape=[
            jax.ShapeDtypeStruct((1024, 128), jnp.float32),
            jax.ShapeDtypeStruct((1024, 128), jnp.float32),
        ],
    )(A0, A1, A2, A3)


def _permute(x, idx):
    dn = lax.GatherDimensionNumbers(
        offset_dims=(), collapsed_slice_dims=(0,), start_index_map=(0,)
    )
    return lax.gather(
        x, idx[:, None], dn, slice_sizes=(1,),
        mode=lax.GatherScatterMode.PROMISE_IN_BOUNDS,
    )


_mesh = plsc.VectorSubcoreMesh(core_axis_name="c", subcore_axis_name="s")


@functools.partial(
    pl.kernel,
    out_type=jax.ShapeDtypeStruct((B, 64), jnp.float32),
    mesh=_mesh,
    scratch_types=[
        pltpu.VMEM((BPW,), jnp.int32),
        pltpu.VMEM((BPW,), jnp.int32),
        pltpu.VMEM((BPW,), jnp.int32),
        pltpu.VMEM((K, 128), jnp.float32),
        pltpu.VMEM((K, 128), jnp.float32),
        pltpu.VMEM((K, 128), jnp.float32),
        pltpu.VMEM((K, 128), jnp.float32),
        pltpu.VMEM((K, 64), jnp.float32),
        pltpu.VMEM((K, 64), jnp.float32),
        pltpu.SemaphoreType.DMA,
        pltpu.SemaphoreType.DMA,
        pltpu.SemaphoreType.DMA,
        pltpu.SemaphoreType.DMA,
    ],
    compiler_params=pltpu.CompilerParams(
        needs_layout_passes=False, use_tc_tiling_on_sc=True
    ),
)
def _sc_lookup(x_hbm, t1_hbm, t2_hbm, out_hbm, xall, ia, ib,
               av0, av1, bv0, bv1, ov0, ov1, sg0, sg1, so0, so1):
    wid = lax.axis_index("s") * 2 + lax.axis_index("c")
    base = wid * BPW
    avs, bvs, ovs = (av0, av1), (bv0, bv1), (ov0, ov1)
    sgs, sos = (sg0, sg1), (so0, so1)

    pltpu.sync_copy(x_hbm.at[pl.ds(base, BPW)], xall)

    def idx_body(g, carry):
        xs = xall[pl.ds(g * 16, 16)]
        ia[pl.ds(g * 16, 16)] = lax.shift_right_logical(xs, 10)
        ib[pl.ds(g * 16, 16)] = lax.bitwise_and(xs, 1023)
        return carry
    lax.fori_loop(0, BPW // 16, idx_body, jnp.int32(0))

    def fire_gathers(ci, s):
        pltpu.async_copy(t1_hbm.at[ia.at[pl.ds(ci * K, K)]], avs[s], sgs[s])
        pltpu.async_copy(t2_hbm.at[ib.at[pl.ds(ci * K, K)]], bvs[s], sgs[s])

    def drain(sem, dst):
        dummy = t1_hbm.at[pl.ds(0, K)] if dst.shape[1] == 128 else out_hbm.at[pl.ds(0, K)]
        pltpu.make_async_copy(dummy, dst, sem).wait()

    def compute_chunk(ci, s):
        av, bv, ov = avs[s], bvs[s], ovs[s]
        drain(sgs[s], av)
        drain(sgs[s], bv)

        def group_body(g, gcarry):
            lanes = lax.iota(jnp.int32, 16)
            rv = g * 16 + lanes
            perms = {s: lax.bitwise_xor(lanes, s) for s in (1, 2, 4, 8)}
            masks = {s: lax.bitwise_and(lanes, s) == 0 for s in (1, 2, 4, 8)}
            for ph in range(2):
                accs = [None] * 32
                for r in range(16):
                    crot = lax.bitwise_and(lanes + r, 15)
                    ap = [
                        plsc.load_gather(av, [rv, crot + (ph * 4 + p) * 16])
                        for p in range(4)
                    ]
                    bq = [
                        plsc.load_gather(bv, [rv, crot + q * 16])
                        for q in range(8)
                    ]
                    for p in range(4):
                        for q in range(8):
                            prod = ap[p] * bq[q]
                            j = p * 8 + q
                            accs[j] = prod if accs[j] is None else accs[j] + prod
                for blk in range(2):
                    v = accs[blk * 16:(blk + 1) * 16]
                    for s in (1, 2, 4, 8):
                        for j in range(16):
                            if j & s:
                                continue
                            a, b = v[j], v[j ^ s]
                            ax = _permute(a, perms[s])
                            bx = _permute(b, perms[s])
                            v[j] = jnp.where(masks[s], a, bx)
                            v[j ^ s] = jnp.where(masks[s], ax, b)
                    base16 = ph * 32 + blk * 16
                    for i in range(16):
                        ov[g * 16 + i, pl.ds(base16, 16)] = v[i]
            return gcarry
        lax.fori_loop(0, NG, group_body, jnp.int32(0))
        pltpu.async_copy(ov, out_hbm.at[pl.ds(base + ci * K, K)], sos[s])

    fire_gathers(0, 0)

    def pair_body(j, carry):
        ci0 = j * 2
        fire_gathers(ci0 + 1, 1)

        @pl.when(j > 0)
        def _():
            drain(sos[0], ov0)
        compute_chunk(ci0, 0)

        @pl.when(ci0 + 2 < NCH)
        def _():
            fire_gathers(ci0 + 2, 0)

        @pl.when(j > 0)
        def _():
            drain(sos[1], ov1)
        compute_chunk(ci0 + 1, 1)
        return carry
    lax.fori_loop(0, NCH // 2, pair_body, jnp.int32(0))
    drain(sos[0], ov0)
    drain(sos[1], ov1)


def kernel(x, U0, U1, U2, U3, ind2coord):
    del ind2coord
    A0 = U0.reshape(64, 16)
    A1 = U1.reshape(16, 2048)
    A3T = U3.reshape(16, 32, 2).transpose(1, 2, 0).reshape(64, 16)
    A2PT = U2.transpose(3, 1, 2, 0).reshape(16, 2048)
    T1, T2 = _make_tables(A0, A1, A3T, A2PT)
    out = _sc_lookup(x.reshape(-1), T1, T2)
    return out.reshape(4096, 26, 64)


# speedup vs baseline: 1.3722x; 1.3722x over previous
import functools

import jax
import jax.numpy as jnp
from jax import lax
from jax.experimental import pallas as pl
from jax.experimental.pallas import tpu as pltpu
from jax.experimental.pallas import tpu_sc as plsc

B = 4096 * 26
NW = 32
BPW = B // NW
K = 128
NCH = BPW // K
NG = K // 16


def _tables_body(a0, a1, a3t, a2pt, t1, t2):
    m1 = jnp.dot(a0[...], a1[...], preferred_element_type=jnp.float32)
    m2q = jnp.dot(a3t[...], a2pt[...], preferred_element_type=jnp.float32)
    t1[...] = (
        m1.reshape(32, 2, 32, 4, 16).transpose(0, 2, 1, 3, 4).reshape(1024, 128)
    )
    t2[...] = (
        m2q.reshape(32, 2, 32, 4, 16).transpose(2, 0, 3, 1, 4).reshape(1024, 128)
    )


def _make_tables(A0, A1, A2, A3):
    return pl.pallas_call(
        _tables_body,
        out_shape=[
            jax.ShapeDtypeStruct((1024, 128), jnp.float32),
            jax.ShapeDtypeStruct((1024, 128), jnp.float32),
        ],
    )(A0, A1, A2, A3)


_mesh = plsc.VectorSubcoreMesh(core_axis_name="c", subcore_axis_name="s")


@functools.partial(
    pl.kernel,
    out_type=jax.ShapeDtypeStruct((B // K, 64, K), jnp.float32),
    mesh=_mesh,
    scratch_types=[
        pltpu.VMEM((BPW,), jnp.int32),
        pltpu.VMEM((BPW,), jnp.int32),
        pltpu.VMEM((BPW,), jnp.int32),
        pltpu.VMEM((K, 128), jnp.float32),
        pltpu.VMEM((K, 128), jnp.float32),
        pltpu.VMEM((K, 128), jnp.float32),
        pltpu.VMEM((K, 128), jnp.float32),
        pltpu.VMEM((64, K), jnp.float32),
        pltpu.VMEM((64, K), jnp.float32),
        pltpu.SemaphoreType.DMA,
        pltpu.SemaphoreType.DMA,
        pltpu.SemaphoreType.DMA,
        pltpu.SemaphoreType.DMA,
    ],
    compiler_params=pltpu.CompilerParams(
        needs_layout_passes=False, use_tc_tiling_on_sc=True
    ),
)
def _sc_lookup(x_hbm, t1_hbm, t2_hbm, out_hbm, xall, ia, ib,
               av0, av1, bv0, bv1, ov0, ov1, sg0, sg1, so0, so1):
    wid = lax.axis_index("s") * 2 + lax.axis_index("c")
    base = wid * BPW
    avs, bvs, ovs = (av0, av1), (bv0, bv1), (ov0, ov1)
    sgs, sos = (sg0, sg1), (so0, so1)

    pltpu.sync_copy(x_hbm.at[pl.ds(base, BPW)], xall)

    def idx_body(g, carry):
        xs = xall[pl.ds(g * 16, 16)]
        ia[pl.ds(g * 16, 16)] = lax.shift_right_logical(xs, 10)
        ib[pl.ds(g * 16, 16)] = lax.bitwise_and(xs, 1023)
        return carry
    lax.fori_loop(0, BPW // 16, idx_body, jnp.int32(0))

    def fire_gathers(ci, s):
        pltpu.async_copy(t1_hbm.at[ia.at[pl.ds(ci * K, K)]], avs[s], sgs[s])
        pltpu.async_copy(t2_hbm.at[ib.at[pl.ds(ci * K, K)]], bvs[s], sgs[s])

    def drain(sem, dst):
        dummy = out_hbm.at[0] if dst.shape[0] == 64 else t1_hbm.at[pl.ds(0, K)]
        pltpu.make_async_copy(dummy, dst, sem).wait()

    def compute_chunk(ci, s):
        av, bv, ov = avs[s], bvs[s], ovs[s]
        drain(sgs[s], av)
        drain(sgs[s], bv)

        def group_body(g, gcarry):
            lanes = lax.iota(jnp.int32, 16)
            rv = g * 16 + lanes
            for ph in range(2):
                accs = [None] * 32
                for r in range(16):
                    crot = lax.bitwise_and(lanes + r, 15)
                    ap = [
                        plsc.load_gather(av, [rv, crot + (ph * 4 + p) * 16])
                        for p in range(4)
                    ]
                    bq = [
                        plsc.load_gather(bv, [rv, crot + q * 16])
                        for q in range(8)
                    ]
                    for p in range(4):
                        for q in range(8):
                            prod = ap[p] * bq[q]
                            j = p * 8 + q
                            accs[j] = prod if accs[j] is None else accs[j] + prod
                for p in range(4):
                    for q in range(8):
                        ov[(ph * 4 + p) * 8 + q, pl.ds(g * 16, 16)] = accs[p * 8 + q]
            return gcarry
        lax.fori_loop(0, NG, group_body, jnp.int32(0))
        pltpu.async_copy(ov, out_hbm.at[wid * NCH + ci], sos[s])

    fire_gathers(0, 0)

    def pair_body(j, carry):
        ci0 = j * 2
        fire_gathers(ci0 + 1, 1)

        @pl.when(j > 0)
        def _():
            drain(sos[0], ov0)
        compute_chunk(ci0, 0)

        @pl.when(ci0 + 2 < NCH)
        def _():
            fire_gathers(ci0 + 2, 0)

        @pl.when(j > 0)
        def _():
            drain(sos[1], ov1)
        compute_chunk(ci0 + 1, 1)
        return carry
    lax.fori_loop(0, NCH // 2, pair_body, jnp.int32(0))
    drain(sos[0], ov0)
    drain(sos[1], ov1)


def kernel(x, U0, U1, U2, U3, ind2coord):
    del ind2coord
    A0 = U0.reshape(64, 16)
    A1 = U1.reshape(16, 2048)
    A3T = U3.reshape(16, 32, 2).transpose(1, 2, 0).reshape(64, 16)
    A2PT = U2.transpose(3, 1, 2, 0).reshape(16, 2048)
    T1, T2 = _make_tables(A0, A1, A3T, A2PT)
    out3 = _sc_lookup(x.reshape(-1), T1, T2)
    out = out3.transpose(0, 2, 1).reshape(B, 64)
    return out.reshape(4096, 26, 64)
